# baseline (speedup 1.0000x reference)
import jax
import jax.numpy as jnp
from jax.experimental import pallas as pl
from jax.experimental.pallas import tpu as pltpu

N = 100000
F = 128
CHUNK = 10000
NCHUNK = N // CHUNK
DEPTH = 3


def _mlp_pipeline_kernel(t_hbm, ws0_ref, bs0_ref, wt0_ref, bt0_ref,
                         ws1_ref, bs1_ref, wt1_ref, bt1_ref, out_hbm,
                         in_bufs, out_bufs, in_sems, out_sems,
                         wmid_ref, bmid_ref):
    dims_nt = (((1,), (1,)), ((), ()))
    dims_nn = (((1,), (0,)), ((), ()))
    bf16 = jnp.bfloat16

    def in_copy(i, slot):
        return pltpu.make_async_copy(
            t_hbm.at[pl.ds(i * CHUNK, CHUNK), :], in_bufs.at[slot],
            in_sems.at[slot])

    def out_copy(i, slot):
        return pltpu.make_async_copy(
            out_bufs.at[slot], out_hbm.at[pl.ds(i * CHUNK, CHUNK), :],
            out_sems.at[slot])

    for k in range(DEPTH):
        in_copy(k, k).start()

    wmid_ref[...] = jax.lax.dot_general(
        ws1_ref[...], wt0_ref[...], dims_nn,
        preferred_element_type=jnp.float32)
    bmid_ref[...] = jax.lax.dot_general(
        bt0_ref[...], ws1_ref[...], dims_nt,
        preferred_element_type=jnp.float32) + bs1_ref[...]

    for i in range(NCHUNK):
        slot = i % DEPTH
        in_copy(i, slot).wait()
        if i >= DEPTH:
            out_copy(i - DEPTH, slot).wait()

        x = in_bufs[slot].astype(bf16)
        h = jax.lax.dot_general(x, ws0_ref[...].astype(bf16), dims_nt,
                                preferred_element_type=jnp.float32)
        h = jnp.maximum(h + bs0_ref[...], 0.0).astype(bf16)
        h = jax.lax.dot_general(h, wmid_ref[...].astype(bf16), dims_nt,
                                preferred_element_type=jnp.float32)
        h = jnp.maximum(h + bmid_ref[...], 0.0).astype(bf16)
        out_bufs[slot] = jax.lax.dot_general(
            h, wt1_ref[...].astype(bf16), dims_nt,
            preferred_element_type=jnp.float32) + bt1_ref[...]

        out_copy(i, slot).start()
        if i + DEPTH < NCHUNK:
            in_copy(i + DEPTH, slot).start()

    for k in range(DEPTH):
        i = NCHUNK - DEPTH + k
        out_copy(i, i % DEPTH).wait()


@jax.jit
def kernel(t, Ws0, bs0, Wt0, bt0, Ws1, bs1, Wt1, bt1):
    weight_spec = pl.BlockSpec((F, F), lambda: (0, 0))
    bias_spec = pl.BlockSpec((1, F), lambda: (0, 0))
    return pl.pallas_call(
        _mlp_pipeline_kernel,
        in_specs=[
            pl.BlockSpec(memory_space=pl.ANY),
            weight_spec, bias_spec,
            weight_spec, bias_spec,
            weight_spec, bias_spec,
            weight_spec, bias_spec,
        ],
        out_specs=pl.BlockSpec(memory_space=pl.ANY),
        out_shape=jax.ShapeDtypeStruct((N, F), jnp.float32),
        scratch_shapes=[
            pltpu.VMEM((DEPTH, CHUNK, F), jnp.float32),
            pltpu.VMEM((DEPTH, CHUNK, F), jnp.float32),
            pltpu.SemaphoreType.DMA((DEPTH,)),
            pltpu.SemaphoreType.DMA((DEPTH,)),
            pltpu.VMEM((F, F), jnp.float32),
            pltpu.VMEM((1, F), jnp.float32),
        ],
    )(t, Ws0, bs0.reshape(1, F), Wt0, bt0.reshape(1, F),
      Ws1, bs1.reshape(1, F), Wt1, bt1.reshape(1, F))


# speedup vs baseline: 1.6567x; 1.6567x over previous
import jax
import jax.numpy as jnp
from jax.experimental import pallas as pl
from jax.experimental.pallas import tpu as pltpu

N = 100000
F = 128
BLOCK = 20000


def _fused_mlp_kernel(t_ref, ws0_ref, bs0_ref, wt0_ref, bt0_ref,
                      ws1_ref, bs1_ref, wt1_ref, bt1_ref, out_ref,
                      wmid_ref, bmid_ref):
    dims_nt = (((1,), (1,)), ((), ()))
    dims_nn = (((1,), (0,)), ((), ()))
    bf16 = jnp.bfloat16

    @pl.when(pl.program_id(0) == 0)
    def _prep():
        wmid_ref[...] = jax.lax.dot_general(
            ws1_ref[...], wt0_ref[...], dims_nn,
            preferred_element_type=jnp.float32)
        bmid_ref[...] = jax.lax.dot_general(
            bt0_ref[...], ws1_ref[...], dims_nt,
            preferred_element_type=jnp.float32) + bs1_ref[...]

    x = t_ref[...].astype(bf16)
    h = jax.lax.dot_general(x, ws0_ref[...].astype(bf16), dims_nt,
                            preferred_element_type=jnp.float32)
    h = jnp.maximum(h.astype(bf16) + bs0_ref[...].astype(bf16), 0.0)
    h = jax.lax.dot_general(h, wmid_ref[...].astype(bf16), dims_nt,
                            preferred_element_type=jnp.float32)
    h = jnp.maximum(h.astype(bf16) + bmid_ref[...].astype(bf16), 0.0)
    out_ref[...] = jax.lax.dot_general(h, wt1_ref[...].astype(bf16), dims_nt,
                                       preferred_element_type=jnp.float32) + bt1_ref[...]


@jax.jit
def kernel(t, Ws0, bs0, Wt0, bt0, Ws1, bs1, Wt1, bt1):
    weight_spec = pl.BlockSpec((F, F), lambda i: (0, 0))
    bias_spec = pl.BlockSpec((1, F), lambda i: (0, 0))
    grid = (N // BLOCK,)
    return pl.pallas_call(
        _fused_mlp_kernel,
        grid=grid,
        in_specs=[
            pl.BlockSpec((BLOCK, F), lambda i: (i, 0)),
            weight_spec, bias_spec,
            weight_spec, bias_spec,
            weight_spec, bias_spec,
            weight_spec, bias_spec,
        ],
        out_specs=pl.BlockSpec((BLOCK, F), lambda i: (i, 0)),
        out_shape=jax.ShapeDtypeStruct((N, F), jnp.float32),
        compiler_params=pltpu.CompilerParams(
            dimension_semantics=("arbitrary",)),
        scratch_shapes=[
            pltpu.VMEM((F, F), jnp.float32),
            pltpu.VMEM((1, F), jnp.float32),
        ],
    )(t, Ws0, bs0.reshape(1, F), Wt0, bt0.reshape(1, F),
      Ws1, bs1.reshape(1, F), Wt1, bt1.reshape(1, F))


# speedup vs baseline: 1.6627x; 1.0036x over previous
import jax
import jax.numpy as jnp
from jax.experimental import pallas as pl
from jax.experimental.pallas import tpu as pltpu

N = 100000
F = 128
BLOCK = 20000


def _fused_mlp_kernel(t_ref, ws0_ref, bs0_ref, wt0_ref, bt0_ref,
                      ws1_ref, bs1_ref, wt1_ref, bt1_ref, out_ref,
                      wmid_ref, bmid_ref):
    dims_nt = (((1,), (1,)), ((), ()))
    dims_nn = (((1,), (0,)), ((), ()))
    bf16 = jnp.bfloat16

    @pl.when(pl.program_id(0) == 0)
    def _prep():
        wmid_ref[...] = jax.lax.dot_general(
            ws1_ref[...], wt0_ref[...], dims_nn,
            preferred_element_type=jnp.float32)
        bmid_ref[...] = jax.lax.dot_general(
            bt0_ref[...], ws1_ref[...], dims_nt,
            preferred_element_type=jnp.float32) + bs1_ref[...]

    x = t_ref[...].astype(bf16)
    h = jax.lax.dot_general(x, ws0_ref[...].astype(bf16), dims_nt,
                            preferred_element_type=jnp.float32)
    h = jnp.maximum(h.astype(bf16) + bs0_ref[...].astype(bf16), 0.0)
    h = jax.lax.dot_general(h, wmid_ref[...].astype(bf16), dims_nt,
                            preferred_element_type=jnp.float32)
    h = jnp.maximum(h.astype(bf16) + bmid_ref[...].astype(bf16), 0.0)
    out_ref[...] = jax.lax.dot_general(h, wt1_ref[...].astype(bf16), dims_nt,
                                       preferred_element_type=jnp.float32) + bt1_ref[...]


@jax.jit
def kernel(t, Ws0, bs0, Wt0, bt0, Ws1, bs1, Wt1, bt1):
    weight_spec = pl.BlockSpec((F, F), lambda i: (0, 0))
    bias_spec = pl.BlockSpec((1, F), lambda i: (0, 0))
    grid = (N // BLOCK,)
    return pl.pallas_call(
        _fused_mlp_kernel,
        grid=grid,
        in_specs=[
            pl.BlockSpec((BLOCK, F), lambda i: (i, 0)),
            weight_spec, bias_spec,
            weight_spec, bias_spec,
            weight_spec, bias_spec,
            weight_spec, bias_spec,
        ],
        out_specs=pl.BlockSpec((BLOCK, F), lambda i: (i, 0)),
        out_shape=jax.ShapeDtypeStruct((N, F), jnp.float32),
        compiler_params=pltpu.CompilerParams(
            dimension_semantics=("parallel",)),
        scratch_shapes=[
            pltpu.VMEM((F, F), jnp.float32),
            pltpu.VMEM((1, F), jnp.float32),
        ],
    )(t, Ws0, bs0.reshape(1, F), Wt0, bt0.reshape(1, F),
      Ws1, bs1.reshape(1, F), Wt1, bt1.reshape(1, F))
